# Initial kernel scaffold; baseline (speedup 1.0000x reference)
import functools

import jax
import jax.numpy as jnp
from jax import lax
from jax.experimental import pallas as pl
from jax.experimental.pallas import tpu as pltpu
from jax.experimental.pallas import tpu_sc as plsc

QN = 1024
NKEYS = 16384
DIM = 128
K_NEG = 10
GAMMA = 1.0
KB = 256
NKB = NKEYS // KB
BIG = jnp.float32(3.0e38)


def _sc_gather_anchors(out1, out2, anchor1, anchor2):
    info = plsc.get_sparse_core_info()
    nc, ns = info.num_cores, info.num_subcores
    nw = nc * ns
    bpw = QN // nw
    mesh = plsc.VectorSubcoreMesh(core_axis_name="c", subcore_axis_name="s")

    @functools.partial(
        pl.kernel,
        mesh=mesh,
        out_type=[
            jax.ShapeDtypeStruct((QN, DIM), jnp.float32),
            jax.ShapeDtypeStruct((QN, DIM), jnp.float32),
        ],
        scratch_types=[
            pltpu.VMEM((bpw,), jnp.int32),
            pltpu.VMEM((bpw, DIM), jnp.float32),
            pltpu.SemaphoreType.DMA,
        ],
    )
    def gather_kernel(t1, t2, i1, i2, o1, o2, idx_v, rows_v, sem):
        wid = lax.axis_index("s") * nc + lax.axis_index("c")
        base = wid * bpw
        pltpu.sync_copy(i1.at[pl.ds(base, bpw)], idx_v)
        pltpu.async_copy(t1.at[idx_v], rows_v, sem).wait()
        pltpu.sync_copy(rows_v, o1.at[pl.ds(base, bpw)])
        pltpu.sync_copy(i2.at[pl.ds(base, bpw)], idx_v)
        pltpu.async_copy(t2.at[idx_v], rows_v, sem).wait()
        pltpu.sync_copy(rows_v, o2.at[pl.ds(base, bpw)])

    return gather_kernel(out1, out2, anchor1, anchor2)


def _topk_merge(x):
    rows = []
    iota = lax.broadcasted_iota(jnp.int32, x.shape, 0)
    for _ in range(K_NEG):
        m = jnp.min(x, axis=0, keepdims=True)
        hit = x <= m
        idx = jnp.min(jnp.where(hit, iota, jnp.int32(1 << 30)), axis=0,
                      keepdims=True)
        x = jnp.where(iota == idx, BIG, x)
        rows.append(m)
    pad = jnp.full((16 - K_NEG, x.shape[1]), BIG, dtype=jnp.float32)
    return jnp.concatenate(rows + [pad], axis=0)


def _side_kernel(b_ref, atq_ref, ato_ref, out_ref, buf_ref):
    kb = pl.program_id(0)

    @pl.when(kb == 0)
    def _init():
        buf_ref[...] = jnp.full((16, QN), BIG, dtype=jnp.float32)

    b = b_ref[...]
    atq = atq_ref[...]
    acc = jnp.zeros((KB, QN), dtype=jnp.float32)
    for d in range(DIM):
        acc = acc + jnp.abs(b[:, d:d + 1] - atq[d:d + 1, :])
    cand = jnp.concatenate([acc, buf_ref[...]], axis=0)
    buf_ref[...] = _topk_merge(cand)

    @pl.when(kb == NKB - 1)
    def _finish():
        avec = jnp.sum(jnp.abs(atq_ref[...] - ato_ref[...]), axis=0,
                       keepdims=True)
        contrib = jnp.maximum(avec + GAMMA - buf_ref[0:K_NEG, :], 0.0)
        total = jnp.sum(contrib, axis=0, keepdims=True)
        out_ref[...] = jnp.sum(total, axis=1, keepdims=True)


def _side_loss(keys, atq, ato):
    return pl.pallas_call(
        _side_kernel,
        grid=(NKB,),
        in_specs=[
            pl.BlockSpec((KB, DIM), lambda kb: (kb, 0)),
            pl.BlockSpec((DIM, QN), lambda kb: (0, 0)),
            pl.BlockSpec((DIM, QN), lambda kb: (0, 0)),
        ],
        out_specs=pl.BlockSpec((1, 1), lambda kb: (0, 0)),
        out_shape=jax.ShapeDtypeStruct((1, 1), jnp.float32),
        scratch_shapes=[pltpu.VMEM((16, QN), jnp.float32)],
    )(keys, atq, ato)


def kernel(out1, out2, anchor1, anchor2):
    a1, a2 = _sc_gather_anchors(out1, out2, anchor1, anchor2)
    at1 = a1.T
    at2 = a2.T
    l1 = _side_loss(out2, at1, at2)
    l2 = _side_loss(out1, at2, at1)
    return (l1[0, 0] + l2[0, 0]) / (QN * K_NEG)


# speedup vs baseline: 3.0259x; 3.0259x over previous
import functools

import jax
import jax.numpy as jnp
from jax import lax
from jax.experimental import pallas as pl
from jax.experimental.pallas import tpu as pltpu
from jax.experimental.pallas import tpu_sc as plsc

QN = 1024
NKEYS = 16384
DIM = 128
K_NEG = 10
GAMMA = 1.0
KB = 256
NKB = NKEYS // KB
BIG = 3.0e38


def _sc_gather_anchors(out1, out2, anchor1, anchor2):
    info = plsc.get_sparse_core_info()
    nc, ns = info.num_cores, info.num_subcores
    nw = nc * ns
    bpw = QN // nw
    mesh = plsc.VectorSubcoreMesh(core_axis_name="c", subcore_axis_name="s")

    @functools.partial(
        pl.kernel,
        mesh=mesh,
        out_type=[
            jax.ShapeDtypeStruct((QN, DIM), jnp.float32),
            jax.ShapeDtypeStruct((QN, DIM), jnp.float32),
        ],
        scratch_types=[
            pltpu.VMEM((bpw,), jnp.int32),
            pltpu.VMEM((bpw, DIM), jnp.float32),
            pltpu.SemaphoreType.DMA,
        ],
    )
    def gather_kernel(t1, t2, i1, i2, o1, o2, idx_v, rows_v, sem):
        wid = lax.axis_index("s") * nc + lax.axis_index("c")
        base = wid * bpw
        pltpu.sync_copy(i1.at[pl.ds(base, bpw)], idx_v)
        pltpu.async_copy(t1.at[idx_v], rows_v, sem).wait()
        pltpu.sync_copy(rows_v, o1.at[pl.ds(base, bpw)])
        pltpu.sync_copy(i2.at[pl.ds(base, bpw)], idx_v)
        pltpu.async_copy(t2.at[idx_v], rows_v, sem).wait()
        pltpu.sync_copy(rows_v, o2.at[pl.ds(base, bpw)])

    return gather_kernel(out1, out2, anchor1, anchor2)


def _topk_merge(x):
    rows = []
    iota = lax.broadcasted_iota(jnp.int32, x.shape, 0)
    for _ in range(K_NEG):
        m = jnp.min(x, axis=0, keepdims=True)
        hit = x <= m
        idx = jnp.min(jnp.where(hit, iota, jnp.int32(1 << 30)), axis=0,
                      keepdims=True)
        x = jnp.where(iota == idx, BIG, x)
        rows.append(m)
    pad = jnp.full((16 - K_NEG, x.shape[1]), BIG, dtype=jnp.float32)
    return jnp.concatenate(rows + [pad], axis=0)


def _side_kernel(b_ref, atq_ref, ato_ref, out_ref, buf_ref):
    kb = pl.program_id(0)

    @pl.when(kb == 0)
    def _init():
        buf_ref[...] = jnp.full((16, QN), BIG, dtype=jnp.float32)

    b = b_ref[...]
    atq = atq_ref[...]
    acc = jnp.zeros((KB, QN), dtype=jnp.float32)
    for d in range(DIM):
        acc = acc + jnp.abs(b[:, d:d + 1] - atq[d:d + 1, :])
    cand = jnp.concatenate([acc, buf_ref[...]], axis=0)
    buf_ref[...] = _topk_merge(cand)

    @pl.when(kb == NKB - 1)
    def _finish():
        avec = jnp.sum(jnp.abs(atq_ref[...] - ato_ref[...]), axis=0,
                       keepdims=True)
        contrib = jnp.maximum(avec + GAMMA - buf_ref[0:K_NEG, :], 0.0)
        total = jnp.sum(contrib, axis=0, keepdims=True)
        out_ref[...] = jnp.sum(total, axis=1, keepdims=True)


def _side_loss(keys, atq, ato):
    return pl.pallas_call(
        _side_kernel,
        grid=(NKB,),
        in_specs=[
            pl.BlockSpec((KB, DIM), lambda kb: (kb, 0)),
            pl.BlockSpec((DIM, QN), lambda kb: (0, 0)),
            pl.BlockSpec((DIM, QN), lambda kb: (0, 0)),
        ],
        out_specs=pl.BlockSpec((1, 1), lambda kb: (0, 0)),
        out_shape=jax.ShapeDtypeStruct((1, 1), jnp.float32),
        scratch_shapes=[pltpu.VMEM((16, QN), jnp.float32)],
    )(keys, atq, ato)


def kernel(out1, out2, anchor1, anchor2):
    a1, a2 = _sc_gather_anchors(out1, out2, anchor1, anchor2)
    at1 = a1.T
    at2 = a2.T
    l1 = _side_loss(out2, at1, at2)
    l2 = _side_loss(out1, at2, at1)
    return (l1[0, 0] + l2[0, 0]) / (QN * K_NEG)


# speedup vs baseline: 3.8655x; 1.2775x over previous
import functools

import jax
import jax.numpy as jnp
from jax import lax
from jax.experimental import pallas as pl
from jax.experimental.pallas import tpu as pltpu
from jax.experimental.pallas import tpu_sc as plsc

QN = 1024
NKEYS = 16384
DIM = 128
K_NEG = 10
GAMMA = 1.0
KB = 256
NKB = NKEYS // KB
KCH = 32
BIG = 3.0e38


def _sc_gather_anchors(out1, out2, anchor1, anchor2):
    info = plsc.get_sparse_core_info()
    nc, ns = info.num_cores, info.num_subcores
    nw = nc * ns
    bpw = QN // nw
    mesh = plsc.VectorSubcoreMesh(core_axis_name="c", subcore_axis_name="s")

    @functools.partial(
        pl.kernel,
        mesh=mesh,
        out_type=[
            jax.ShapeDtypeStruct((QN, DIM), jnp.float32),
            jax.ShapeDtypeStruct((QN, DIM), jnp.float32),
        ],
        scratch_types=[
            pltpu.VMEM((bpw,), jnp.int32),
            pltpu.VMEM((bpw, DIM), jnp.float32),
            pltpu.SemaphoreType.DMA,
        ],
    )
    def gather_kernel(t1, t2, i1, i2, o1, o2, idx_v, rows_v, sem):
        wid = lax.axis_index("s") * nc + lax.axis_index("c")
        base = wid * bpw
        pltpu.sync_copy(i1.at[pl.ds(base, bpw)], idx_v)
        pltpu.async_copy(t1.at[idx_v], rows_v, sem).wait()
        pltpu.sync_copy(rows_v, o1.at[pl.ds(base, bpw)])
        pltpu.sync_copy(i2.at[pl.ds(base, bpw)], idx_v)
        pltpu.async_copy(t2.at[idx_v], rows_v, sem).wait()
        pltpu.sync_copy(rows_v, o2.at[pl.ds(base, bpw)])

    return gather_kernel(out1, out2, anchor1, anchor2)


def _topk_merge(x):
    rows = []
    iota = lax.broadcasted_iota(jnp.int32, x.shape, 0)
    for _ in range(K_NEG):
        m = jnp.min(x, axis=0, keepdims=True)
        hit = x <= m
        idx = jnp.min(jnp.where(hit, iota, jnp.int32(1 << 30)), axis=0,
                      keepdims=True)
        x = jnp.where(iota == idx, BIG, x)
        rows.append(m)
    pad = jnp.full((16 - K_NEG, x.shape[1]), BIG, dtype=jnp.float32)
    return jnp.concatenate(rows + [pad], axis=0)


def _side_kernel(b_ref, atq_ref, ato_ref, out_ref, buf_ref):
    kb = pl.program_id(0)

    @pl.when(kb == 0)
    def _init():
        buf_ref[...] = jnp.full((16, QN), BIG, dtype=jnp.float32)

    b = b_ref[...]
    atq = atq_ref[...]
    rs_b = jnp.sum(b, axis=1, keepdims=True)
    rs_a = jnp.sum(atq, axis=0, keepdims=True)
    parts = []
    for kt in range(KB // KCH):
        bch = b[kt * KCH:(kt + 1) * KCH, :]
        ac = jnp.minimum(bch[:, 0:1], atq[0:1, :])
        for d in range(1, DIM):
            ac = ac + jnp.minimum(bch[:, d:d + 1], atq[d:d + 1, :])
        parts.append(ac)
    mins = jnp.concatenate(parts, axis=0)
    acc = (rs_b + rs_a) - (mins + mins)
    cand = jnp.concatenate([acc, buf_ref[...]], axis=0)
    buf_ref[...] = _topk_merge(cand)

    @pl.when(kb == NKB - 1)
    def _finish():
        avec = jnp.sum(jnp.abs(atq_ref[...] - ato_ref[...]), axis=0,
                       keepdims=True)
        contrib = jnp.maximum(avec + GAMMA - buf_ref[0:K_NEG, :], 0.0)
        total = jnp.sum(contrib, axis=0, keepdims=True)
        out_ref[...] = jnp.sum(total, axis=1, keepdims=True)


def _side_loss(keys, atq, ato):
    return pl.pallas_call(
        _side_kernel,
        grid=(NKB,),
        in_specs=[
            pl.BlockSpec((KB, DIM), lambda kb: (kb, 0)),
            pl.BlockSpec((DIM, QN), lambda kb: (0, 0)),
            pl.BlockSpec((DIM, QN), lambda kb: (0, 0)),
        ],
        out_specs=pl.BlockSpec((1, 1), lambda kb: (0, 0)),
        out_shape=jax.ShapeDtypeStruct((1, 1), jnp.float32),
        scratch_shapes=[pltpu.VMEM((16, QN), jnp.float32)],
    )(keys, atq, ato)


def kernel(out1, out2, anchor1, anchor2):
    a1, a2 = _sc_gather_anchors(out1, out2, anchor1, anchor2)
    at1 = a1.T
    at2 = a2.T
    l1 = _side_loss(out2, at1, at2)
    l2 = _side_loss(out1, at2, at1)
    return (l1[0, 0] + l2[0, 0]) / (QN * K_NEG)
